# baseline (speedup 1.0000x reference)
import functools

import jax
import jax.numpy as jnp
from jax import lax
from jax.experimental import pallas as pl
from jax.experimental.pallas import tpu as pltpu
from jax.experimental.pallas import tpu_sc as plsc

NCLUSTER = 512
NB_SC = 8
PTS = 2048
NW = 32


def _sc_body(x0_hbm, x1_hbm, x2_hbm, w0_hbm, w1_hbm, w2_hbm, w3_hbm,
             out_hbm, xv0, xv1, xv2, w0v, w1v, w2v, w3v, ov):
    ppw = x0_hbm.shape[0] // NW
    wid = lax.axis_index("s") * 2 + lax.axis_index("c")
    pltpu.sync_copy(w0_hbm, w0v)
    pltpu.sync_copy(w1_hbm, w1v)
    pltpu.sync_copy(w2_hbm, w2v)
    pltpu.sync_copy(w3_hbm, w3v)
    pltpu.sync_copy(x0_hbm.at[pl.ds(wid * ppw, ppw)], xv0)
    pltpu.sync_copy(x1_hbm.at[pl.ds(wid * ppw, ppw)], xv1)
    pltpu.sync_copy(x2_hbm.at[pl.ds(wid * ppw, ppw)], xv2)

    def quad(q, carry):
        xp = [
            (
                xv0[pl.ds(q * 64 + g * 16, 16)],
                xv1[pl.ds(q * 64 + g * 16, 16)],
                xv2[pl.ds(q * 64 + g * 16, 16)],
            )
            for g in range(4)
        ]

        def cl(k, bb):
            bests, bidxs = bb
            w0 = w0v[pl.ds(k * 16, 16)]
            w1 = w1v[pl.ds(k * 16, 16)]
            w2 = w2v[pl.ds(k * 16, 16)]
            cns = w3v[pl.ds(k * 16, 16)]
            nb, ni = [], []
            for g in range(4):
                xp0, xp1, xp2 = xp[g]
                s = ((cns + w0 * xp0) + w1 * xp1) + w2 * xp2
                m = s < bests[g]
                nb.append(jnp.where(m, s, bests[g]))
                ni.append(jnp.where(m, k, bidxs[g]))
            return tuple(nb), tuple(ni)

        inf = jnp.full((16,), jnp.inf, jnp.float32)
        zero = jnp.zeros((16,), jnp.int32)
        _, bidxs = lax.fori_loop(
            0, NCLUSTER, cl, ((inf,) * 4, (zero,) * 4), unroll=4
        )
        for g in range(4):
            ov[pl.ds(q * 64 + g * 16, 16)] = bidxs[g]
        return carry

    lax.fori_loop(0, ppw // 64, quad, 0)
    pltpu.sync_copy(ov, out_hbm.at[pl.ds(wid * ppw, ppw)])


def _sc_assign(xr_sc, wbc):
    nb, _, hw = xr_sc.shape
    n = nb * hw
    x0f = xr_sc[:, 0, :].reshape(n)
    x1f = xr_sc[:, 1, :].reshape(n)
    x2f = xr_sc[:, 2, :].reshape(n)
    ppw = n // NW
    mesh = plsc.VectorSubcoreMesh(core_axis_name="c", subcore_axis_name="s")
    fn = functools.partial(
        pl.kernel,
        mesh=mesh,
        out_type=jax.ShapeDtypeStruct((n,), jnp.int32),
        scratch_types=[
            pltpu.VMEM((ppw,), jnp.float32),
            pltpu.VMEM((ppw,), jnp.float32),
            pltpu.VMEM((ppw,), jnp.float32),
            pltpu.VMEM((NCLUSTER * 16,), jnp.float32),
            pltpu.VMEM((NCLUSTER * 16,), jnp.float32),
            pltpu.VMEM((NCLUSTER * 16,), jnp.float32),
            pltpu.VMEM((NCLUSTER * 16,), jnp.float32),
            pltpu.VMEM((ppw,), jnp.int32),
        ],
    )(_sc_body)
    return fn(x0f, x1f, x2f, wbc[0], wbc[1], wbc[2], wbc[3])


def _tc_body(x_ref, w_ref, out_ref):
    x0 = x_ref[0, 0:1, :]
    x1 = x_ref[0, 1:2, :]
    x2 = x_ref[0, 2:3, :]
    w0 = w_ref[:, 0:1]
    w1 = w_ref[:, 1:2]
    w2 = w_ref[:, 2:3]
    cn = w_ref[:, 3:4]
    s = ((cn + w0 * x0) + w1 * x1) + w2 * x2
    a = jnp.argmin(s, axis=0).astype(jnp.int32)
    out_ref[0, 0, 0, :] = a


def _tc_assign(xr_tc, wc4):
    nb, c, hw = xr_tc.shape
    nj = hw // PTS
    out = pl.pallas_call(
        _tc_body,
        grid=(nb, nj),
        in_specs=[
            pl.BlockSpec((1, c, PTS), lambda i, j: (i, 0, j)),
            pl.BlockSpec((NCLUSTER, c + 1), lambda i, j: (0, 0)),
        ],
        out_specs=pl.BlockSpec((1, 1, 1, PTS), lambda i, j: (i, j, 0, 0)),
        out_shape=jax.ShapeDtypeStruct((nb, nj, 1, PTS), jnp.int32),
    )(xr_tc, wc4)
    return out.reshape(nb, hw)


def kernel(x, C):
    bs, c, h, w = x.shape
    hw = h * w
    xr = x.reshape(bs, c, hw)
    wc = jnp.concatenate(
        [-2.0 * C.T, (C * C).sum(1, keepdims=True).T], axis=0
    )
    wbc = jnp.broadcast_to(wc[:, :, None], (4, NCLUSTER, 16)).reshape(
        4, NCLUSTER * 16
    )
    out_sc = _sc_assign(xr[:NB_SC], wbc).reshape(NB_SC, hw)
    out_tc = _tc_assign(xr[NB_SC:], wc.T)
    return jnp.concatenate([out_sc, out_tc], axis=0)


# speedup vs baseline: 1.3934x; 1.3934x over previous
import functools

import jax
import jax.numpy as jnp
from jax import lax
from jax.experimental import pallas as pl
from jax.experimental.pallas import tpu as pltpu
from jax.experimental.pallas import tpu_sc as plsc

NCLUSTER = 512
NB_SC = 4
PTS = 2048
NW = 32


def _sc_body(x0_hbm, x1_hbm, x2_hbm, w0_hbm, w1_hbm, w2_hbm, w3_hbm,
             out_hbm, xv0, xv1, xv2, w0v, w1v, w2v, w3v, ov):
    ppw = x0_hbm.shape[0] // NW
    wid = lax.axis_index("s") * 2 + lax.axis_index("c")
    pltpu.sync_copy(w0_hbm, w0v)
    pltpu.sync_copy(w1_hbm, w1v)
    pltpu.sync_copy(w2_hbm, w2v)
    pltpu.sync_copy(w3_hbm, w3v)
    pltpu.sync_copy(x0_hbm.at[pl.ds(wid * ppw, ppw)], xv0)
    pltpu.sync_copy(x1_hbm.at[pl.ds(wid * ppw, ppw)], xv1)
    pltpu.sync_copy(x2_hbm.at[pl.ds(wid * ppw, ppw)], xv2)

    def quad(q, carry):
        xp = [
            (
                xv0[pl.ds(q * 64 + g * 16, 16)],
                xv1[pl.ds(q * 64 + g * 16, 16)],
                xv2[pl.ds(q * 64 + g * 16, 16)],
            )
            for g in range(4)
        ]

        def cl(k, bb):
            bests, bidxs = bb
            w0 = w0v[pl.ds(k * 16, 16)]
            w1 = w1v[pl.ds(k * 16, 16)]
            w2 = w2v[pl.ds(k * 16, 16)]
            cns = w3v[pl.ds(k * 16, 16)]
            nb, ni = [], []
            for g in range(4):
                xp0, xp1, xp2 = xp[g]
                s = ((cns + w0 * xp0) + w1 * xp1) + w2 * xp2
                m = s < bests[g]
                nb.append(jnp.where(m, s, bests[g]))
                ni.append(jnp.where(m, k, bidxs[g]))
            return tuple(nb), tuple(ni)

        inf = jnp.full((16,), jnp.inf, jnp.float32)
        zero = jnp.zeros((16,), jnp.int32)
        _, bidxs = lax.fori_loop(
            0, NCLUSTER, cl, ((inf,) * 4, (zero,) * 4), unroll=4
        )
        for g in range(4):
            ov[pl.ds(q * 64 + g * 16, 16)] = bidxs[g]
        return carry

    lax.fori_loop(0, ppw // 64, quad, 0)
    pltpu.sync_copy(ov, out_hbm.at[pl.ds(wid * ppw, ppw)])


def _sc_assign(xr_sc, wbc):
    nb, _, hw = xr_sc.shape
    n = nb * hw
    x0f = xr_sc[:, 0, :].reshape(n)
    x1f = xr_sc[:, 1, :].reshape(n)
    x2f = xr_sc[:, 2, :].reshape(n)
    ppw = n // NW
    mesh = plsc.VectorSubcoreMesh(core_axis_name="c", subcore_axis_name="s")
    fn = functools.partial(
        pl.kernel,
        mesh=mesh,
        out_type=jax.ShapeDtypeStruct((n,), jnp.int32),
        scratch_types=[
            pltpu.VMEM((ppw,), jnp.float32),
            pltpu.VMEM((ppw,), jnp.float32),
            pltpu.VMEM((ppw,), jnp.float32),
            pltpu.VMEM((NCLUSTER * 16,), jnp.float32),
            pltpu.VMEM((NCLUSTER * 16,), jnp.float32),
            pltpu.VMEM((NCLUSTER * 16,), jnp.float32),
            pltpu.VMEM((NCLUSTER * 16,), jnp.float32),
            pltpu.VMEM((ppw,), jnp.int32),
        ],
    )(_sc_body)
    return fn(x0f, x1f, x2f, wbc[0], wbc[1], wbc[2], wbc[3])


def _tc_body(x_ref, w_ref, out_ref):
    x0 = x_ref[0, 0:1, :]
    x1 = x_ref[0, 1:2, :]
    x2 = x_ref[0, 2:3, :]
    w0 = w_ref[:, 0:1]
    w1 = w_ref[:, 1:2]
    w2 = w_ref[:, 2:3]
    cn = w_ref[:, 3:4]
    s = ((cn + w0 * x0) + w1 * x1) + w2 * x2
    a = jnp.argmin(s, axis=0).astype(jnp.int32)
    out_ref[0, 0, 0, :] = a


def _tc_assign(xr_tc, wc4):
    nb, c, hw = xr_tc.shape
    nj = hw // PTS
    out = pl.pallas_call(
        _tc_body,
        grid=(nb, nj),
        in_specs=[
            pl.BlockSpec((1, c, PTS), lambda i, j: (i, 0, j)),
            pl.BlockSpec((NCLUSTER, c + 1), lambda i, j: (0, 0)),
        ],
        out_specs=pl.BlockSpec((1, 1, 1, PTS), lambda i, j: (i, j, 0, 0)),
        out_shape=jax.ShapeDtypeStruct((nb, nj, 1, PTS), jnp.int32),
    )(xr_tc, wc4)
    return out.reshape(nb, hw)


def kernel(x, C):
    bs, c, h, w = x.shape
    hw = h * w
    xr = x.reshape(bs, c, hw)
    wc = jnp.concatenate(
        [-2.0 * C.T, (C * C).sum(1, keepdims=True).T], axis=0
    )
    wbc = jnp.broadcast_to(wc[:, :, None], (4, NCLUSTER, 16)).reshape(
        4, NCLUSTER * 16
    )
    out_sc = _sc_assign(xr[:NB_SC], wbc).reshape(NB_SC, hw)
    out_tc = _tc_assign(xr[NB_SC:], wc.T)
    return jnp.concatenate([out_sc, out_tc], axis=0)
